# Initial kernel scaffold; baseline (speedup 1.0000x reference)
import jax
import jax.numpy as jnp
from jax.experimental import pallas as pl


def kernel(pillar_features, voxel_coords):
    raise NotImplementedError("write your pallas kernel here")


# speedup vs baseline: 1.5594x; 1.5594x over previous
import functools

import jax
import jax.numpy as jnp
from jax import lax
from jax.experimental import pallas as pl
from jax.experimental.pallas import tpu as pltpu
from jax.experimental.pallas import tpu_sc as plsc

NXc, NYc, Cc, Bc, Pc = 512, 512, 64, 4, 40000
CPT = 32768
LOG2_CPT = 15
NREG = 64
RCELLS = NXc
CHUNK = 2000
NCHUNK = Pc // CHUNK
DUMP = CPT


def _iota16():
    return lax.iota(jnp.int32, 16)


def _popcount(mask):
    return jnp.sum(mask.astype(jnp.int32))


def _sload(ref, i):
    return ref[pl.ds(i, 16)][0]


def _sstore(ref, i, v):
    plsc.store_scatter(ref, [jnp.full((16,), i, jnp.int32)],
                       jnp.full((16,), v, jnp.int32), mask=_iota16() == 0)


def _body(f128_hbm, b_hbm, y_hbm, x_hbm, out_hbm,
          cbuf, rawl, wmap, staging, gbuf, idxb, boff, pos):
    wid = lax.axis_index("s") * 2 + lax.axis_index("c")

    neg1 = jnp.full((16,), -1, jnp.int32)

    @pl.loop(0, CPT, step=16)
    def _(i):
        wmap[pl.ds(i, 16)] = neg1

    def a1_chunk(ci, count):
        base = ci * CHUNK
        pltpu.sync_copy(b_hbm.at[pl.ds(base, CHUNK)],
                        cbuf.at[pl.ds(0, CHUNK)])
        pltpu.sync_copy(y_hbm.at[pl.ds(base, CHUNK)],
                        cbuf.at[pl.ds(CHUNK, CHUNK)])
        pltpu.sync_copy(x_hbm.at[pl.ds(base, CHUNK)],
                        cbuf.at[pl.ds(2 * CHUNK, CHUNK)])

        def vloop(vi, cnt):
            off = vi * 16
            bv = cbuf[pl.ds(off, 16)]
            yv = cbuf[pl.ds(CHUNK + off, 16)]
            xv = cbuf[pl.ds(2 * CHUNK + off, 16)]
            cell = bv * (NYc * NXc) + yv * NXc + xv
            valid = lax.shift_right_logical(cell, LOG2_CPT) == wid
            local = lax.bitwise_and(cell, CPT - 1)
            p = base + off + _iota16()

            wv = plsc.load_gather(wmap, [local])
            m0 = valid & (p > wv)

            def wcond(m):
                return jnp.any(m)

            def wbody(m):
                plsc.store_scatter(wmap, [local], p, mask=m)
                w2 = plsc.load_gather(wmap, [local])
                return valid & (p > w2)

            lax.while_loop(wcond, wbody, m0)

            packed = lax.bitwise_or(local, lax.shift_left(p, LOG2_CPT))
            plsc.store_compressed(rawl.at[pl.ds(cnt, 16)], packed, mask=valid)
            return cnt + _popcount(valid)

        return lax.fori_loop(0, CHUNK // 16, vloop, count)

    na = lax.fori_loop(0, NCHUNK, a1_chunk, jnp.int32(0))

    @pl.loop(0, NREG)
    def _(r):
        pos[r] = 0

    def count_body(e, _):
        pk = _sload(rawl, e)
        local = lax.bitwise_and(pk, CPT - 1)
        p = lax.shift_right_logical(pk, LOG2_CPT)
        live_i = (_sload(wmap, local) == p).astype(jnp.int32)
        rg = lax.shift_right_logical(local, 9)
        pos[rg] += live_i
        _sstore(rawl, e, lax.bitwise_or(pk, lax.shift_left(live_i, 31)))
        return 0

    lax.fori_loop(0, na, count_body, 0)

    def prefix_body(r, acc):
        c = pos[r]
        boff[r] = acc
        pos[r] = acc
        return acc + c

    nb = lax.fori_loop(0, NREG, prefix_body, jnp.int32(0))
    boff[NREG] = nb

    def place_body(e, _):
        pk = _sload(rawl, e)
        live = pk < 0
        pkc = lax.bitwise_and(pk, 0x7FFFFFFF)
        rg = lax.shift_right_logical(lax.bitwise_and(pkc, CPT - 1), 9)
        o = pos[rg]
        dest = jnp.where(live, o, DUMP)
        _sstore(wmap, dest, pkc)
        pos[rg] = o + live.astype(jnp.int32)
        return 0

    lax.fori_loop(0, na, place_body, 0)

    zero16 = jnp.zeros((16,), jnp.float32)
    chanbase = (wid >> 3) * Cc
    colbase = lax.bitwise_and(wid, 7) * CPT

    @pl.loop(0, Cc)
    def _(c):
        @pl.loop(0, RCELLS, step=16)
        def _(j):
            staging[c, pl.ds(j, 16)] = zero16

    def region_body(r, _):
        start = boff[r]
        end = boff[r + 1]
        nch = (end - start + 15) >> 4

        def chunk_body(ch, _):
            cstart = start + ch * 16
            pk = wmap[pl.ds(cstart, 16)]
            okm = (cstart + _iota16()) < end
            rid = jnp.where(okm, lax.shift_right_logical(pk, LOG2_CPT + 1), 0)
            idxb[pl.ds(0, 16)] = rid
            pltpu.sync_copy(f128_hbm.at[idxb], gbuf)
            k = jnp.minimum(jnp.int32(16), end - cstart)

            def place(j, _):
                pkj = _sload(wmap, cstart + j)
                lr = lax.bitwise_and(pkj, RCELLS - 1)
                pj = lax.shift_right_logical(pkj, LOG2_CPT)
                half = lax.bitwise_and(pj, 1) * Cc
                lr_s = jnp.full((16,), lr, jnp.int32)
                for q in range(4):
                    vals = gbuf[j, pl.ds(half + q * 16, 16)]
                    plsc.store_scatter(staging, [q * 16 + _iota16(), lr_s], vals)
                return 0

            lax.fori_loop(0, k, place, 0)
            return 0

        lax.fori_loop(0, nch, chunk_body, 0)

        pltpu.sync_copy(staging,
                        out_hbm.at[pl.ds(chanbase, Cc),
                                   pl.ds(colbase + r * RCELLS, RCELLS)])

        def clean(e, _):
            pkj = _sload(wmap, e)
            lr = lax.bitwise_and(pkj, RCELLS - 1)
            lr_s = jnp.full((16,), lr, jnp.int32)
            for q in range(4):
                plsc.store_scatter(staging, [q * 16 + _iota16(), lr_s], zero16)
            return 0

        lax.fori_loop(start, end, clean, 0)
        return 0

    lax.fori_loop(0, NREG, region_body, 0)


@jax.jit
def kernel(pillar_features, voxel_coords):
    f128 = pillar_features.reshape(Pc // 2, 2 * Cc)
    cols = voxel_coords.T
    bcol = cols[0]
    ycol = cols[2]
    xcol = cols[3]

    mesh = plsc.VectorSubcoreMesh(core_axis_name="c", subcore_axis_name="s")
    run = pl.kernel(
        _body,
        out_type=jax.ShapeDtypeStruct((Bc * Cc, NYc * NXc), jnp.float32),
        mesh=mesh,
        scratch_types=[
            pltpu.VMEM((3 * CHUNK,), jnp.int32),
            pltpu.VMEM((Pc,), jnp.int32),
            pltpu.VMEM((CPT + 32,), jnp.int32),
            pltpu.VMEM((Cc, RCELLS), jnp.float32),
            pltpu.VMEM((16, 2 * Cc), jnp.float32),
            pltpu.VMEM((16,), jnp.int32),
            pltpu.SMEM((NREG + 1,), jnp.int32),
            pltpu.SMEM((NREG,), jnp.int32),
        ],
        compiler_params=pltpu.CompilerParams(needs_layout_passes=False),
    )
    out_flat = run(f128, bcol, ycol, xcol)
    return out_flat.reshape(Bc, Cc, NYc, NXc)


# speedup vs baseline: 1.8134x; 1.1629x over previous
import functools

import jax
import jax.numpy as jnp
from jax import lax
from jax.experimental import pallas as pl
from jax.experimental.pallas import tpu as pltpu
from jax.experimental.pallas import tpu_sc as plsc

NXc, NYc, Cc, Bc, Pc = 512, 512, 64, 4, 40000
CPT = 32768
LOG2_CPT = 15
NREG = 64
RCELLS = NXc
CHUNK = 2000
NCHUNK = Pc // CHUNK
DUMP = CPT


def _iota16():
    return lax.iota(jnp.int32, 16)


def _popcount(mask):
    return jnp.sum(mask.astype(jnp.int32))


def _sload(ref, i):
    return ref[pl.ds(i, 16)][0]


def _sstore(ref, i, v):
    plsc.store_scatter(ref, [jnp.full((16,), i, jnp.int32)],
                       jnp.full((16,), v, jnp.int32), mask=_iota16() == 0)


def _body(f128_hbm, b_hbm, y_hbm, x_hbm, out_hbm,
          cbuf, rawl, wmap, staging, gbuf, idxb, boff, pos):
    wid = lax.axis_index("s") * 2 + lax.axis_index("c")

    neg1 = jnp.full((16,), -1, jnp.int32)

    @pl.loop(0, CPT, step=16)
    def _(i):
        wmap[pl.ds(i, 16)] = neg1

    def a1_chunk(ci, count):
        base = ci * CHUNK
        pltpu.sync_copy(b_hbm.at[pl.ds(base, CHUNK)],
                        cbuf.at[pl.ds(0, CHUNK)])
        pltpu.sync_copy(y_hbm.at[pl.ds(base, CHUNK)],
                        cbuf.at[pl.ds(CHUNK, CHUNK)])
        pltpu.sync_copy(x_hbm.at[pl.ds(base, CHUNK)],
                        cbuf.at[pl.ds(2 * CHUNK, CHUNK)])

        def vloop(vi, cnt):
            off = vi * 16
            bv = cbuf[pl.ds(off, 16)]
            yv = cbuf[pl.ds(CHUNK + off, 16)]
            xv = cbuf[pl.ds(2 * CHUNK + off, 16)]
            cell = bv * (NYc * NXc) + yv * NXc + xv
            valid = lax.shift_right_logical(cell, LOG2_CPT) == wid
            local = lax.bitwise_and(cell, CPT - 1)
            p = base + off + _iota16()

            wv = plsc.load_gather(wmap, [local])
            m0 = valid & (p > wv)

            def wcond(m):
                return jnp.any(m)

            def wbody(m):
                plsc.store_scatter(wmap, [local], p, mask=m)
                w2 = plsc.load_gather(wmap, [local])
                return valid & (p > w2)

            lax.while_loop(wcond, wbody, m0)

            packed = lax.bitwise_or(local, lax.shift_left(p, LOG2_CPT))
            plsc.store_compressed(rawl.at[pl.ds(cnt, 16)], packed, mask=valid)
            return cnt + _popcount(valid)

        return lax.fori_loop(0, CHUNK // 16, vloop, count)

    na = lax.fori_loop(0, NCHUNK, a1_chunk, jnp.int32(0))

    @pl.loop(0, NREG)
    def _(r):
        pos[r] = 0

    def count_body(e, _):
        pk = _sload(rawl, e)
        local = lax.bitwise_and(pk, CPT - 1)
        p = lax.shift_right_logical(pk, LOG2_CPT)
        live_i = (_sload(wmap, local) == p).astype(jnp.int32)
        rg = lax.shift_right_logical(local, 9)
        pos[rg] += live_i
        _sstore(rawl, e, lax.bitwise_or(pk, lax.shift_left(live_i, 31)))
        return 0

    lax.fori_loop(0, na, count_body, 0)

    def prefix_body(r, acc):
        c = pos[r]
        boff[r] = acc
        pos[r] = acc
        return acc + c

    nb = lax.fori_loop(0, NREG, prefix_body, jnp.int32(0))
    boff[NREG] = nb

    def place_body(e, _):
        pk = _sload(rawl, e)
        live = pk < 0
        pkc = lax.bitwise_and(pk, 0x7FFFFFFF)
        rg = lax.shift_right_logical(lax.bitwise_and(pkc, CPT - 1), 9)
        o = pos[rg]
        dest = jnp.where(live, o, DUMP)
        _sstore(wmap, dest, pkc)
        pos[rg] = o + live.astype(jnp.int32)
        return 0

    lax.fori_loop(0, na, place_body, 0)

    zero16 = jnp.zeros((16,), jnp.float32)
    chanbase = (wid >> 3) * Cc
    yrowbase = lax.bitwise_and(wid, 7) * NREG

    @pl.loop(0, Cc)
    def _(c):
        @pl.loop(0, RCELLS, step=16)
        def _(j):
            staging[c, pl.ds(j, 16)] = zero16

    def region_body(r, _):
        start = boff[r]
        end = boff[r + 1]
        nch = (end - start + 15) >> 4

        def chunk_body(ch, _):
            cstart = start + ch * 16
            pk = wmap[pl.ds(cstart, 16)]
            okm = (cstart + _iota16()) < end
            rid = jnp.where(okm, lax.shift_right_logical(pk, LOG2_CPT + 1), 0)
            idxb[pl.ds(0, 16)] = rid
            pltpu.sync_copy(f128_hbm.at[idxb], gbuf)
            k = jnp.minimum(jnp.int32(16), end - cstart)

            def place(j, _):
                pkj = _sload(wmap, cstart + j)
                lr = lax.bitwise_and(pkj, RCELLS - 1)
                pj = lax.shift_right_logical(pkj, LOG2_CPT)
                half = lax.bitwise_and(pj, 1) * Cc
                lr_s = jnp.full((16,), lr, jnp.int32)
                for q in range(4):
                    vals = gbuf[j, pl.ds(half + q * 16, 16)]
                    plsc.store_scatter(staging, [q * 16 + _iota16(), lr_s], vals)
                return 0

            lax.fori_loop(0, k, place, 0)
            return 0

        lax.fori_loop(0, nch, chunk_body, 0)

        pltpu.sync_copy(staging,
                        out_hbm.at[pl.ds(chanbase, Cc),
                                   yrowbase + r,
                                   pl.ds(0, NXc)])

        def clean(e, _):
            pkj = _sload(wmap, e)
            lr = lax.bitwise_and(pkj, RCELLS - 1)
            lr_s = jnp.full((16,), lr, jnp.int32)
            for q in range(4):
                plsc.store_scatter(staging, [q * 16 + _iota16(), lr_s], zero16)
            return 0

        lax.fori_loop(start, end, clean, 0)
        return 0

    lax.fori_loop(0, NREG, region_body, 0)


@jax.jit
def kernel(pillar_features, voxel_coords):
    f128 = pillar_features.reshape(Pc // 2, 2 * Cc)
    cols = voxel_coords.T
    bcol = cols[0]
    ycol = cols[2]
    xcol = cols[3]

    mesh = plsc.VectorSubcoreMesh(core_axis_name="c", subcore_axis_name="s")
    run = pl.kernel(
        _body,
        out_type=jax.ShapeDtypeStruct((Bc * Cc, NYc, NXc), jnp.float32),
        mesh=mesh,
        scratch_types=[
            pltpu.VMEM((3 * CHUNK,), jnp.int32),
            pltpu.VMEM((Pc,), jnp.int32),
            pltpu.VMEM((CPT + 32,), jnp.int32),
            pltpu.VMEM((Cc, RCELLS), jnp.float32),
            pltpu.VMEM((16, 2 * Cc), jnp.float32),
            pltpu.VMEM((16,), jnp.int32),
            pltpu.SMEM((NREG + 1,), jnp.int32),
            pltpu.SMEM((NREG,), jnp.int32),
        ],
        compiler_params=pltpu.CompilerParams(needs_layout_passes=False),
    )
    out_flat = run(f128, bcol, ycol, xcol)
    return out_flat.reshape(Bc, Cc, NYc, NXc)
